# baseline (speedup 1.0000x reference)
import jax
import jax.numpy as jnp
from jax.experimental import pallas as pl
from jax.experimental.pallas import tpu as pltpu

_D_MODEL = 384
_D_INNER = 768
_D_STATE = 16
_DT_RANK = 24
_D_CONV = 4
_VOCAB = 50257
_B = 2
_LA = 1024
_LT = 1024
_L = _LA + _LT

_T = 128
_NC = _L // _T
_NV = 1024
_VMEM = 56 * 1024 * 1024


def _audio_kernel(x_ref, w1_ref, b1_ref, w2_ref, b2_ref, o_ref):
    x = x_ref[0]
    zr1 = jnp.zeros((1, x.shape[1]), jnp.float32)
    xp = jnp.concatenate([zr1, x[:-1]], axis=0)
    xn = jnp.concatenate([x[1:], zr1], axis=0)
    y1 = (jnp.dot(xp, w1_ref[0], preferred_element_type=jnp.float32)
          + jnp.dot(x, w1_ref[1], preferred_element_type=jnp.float32)
          + jnp.dot(xn, w1_ref[2], preferred_element_type=jnp.float32))
    y1 = jnp.maximum(y1 + b1_ref[...], 0.0)
    zr2 = jnp.zeros((1, y1.shape[1]), jnp.float32)
    yp = jnp.concatenate([zr2, y1[:-1]], axis=0)
    yn = jnp.concatenate([y1[1:], zr2], axis=0)
    y2 = (jnp.dot(yp, w2_ref[0], preferred_element_type=jnp.float32)
          + jnp.dot(y1, w2_ref[1], preferred_element_type=jnp.float32)
          + jnp.dot(yn, w2_ref[2], preferred_element_type=jnp.float32))
    o_ref[0] = y2 + b2_ref[...]


def _audio_frontend(audio_t, w1t, b1, w2t, b2):
    return pl.pallas_call(
        _audio_kernel,
        out_shape=jax.ShapeDtypeStruct((_B, _LA, _D_MODEL), jnp.float32),
        grid=(_B,),
        in_specs=[
            pl.BlockSpec((1, _LA, 80), lambda b: (b, 0, 0)),
            pl.BlockSpec((3, 80, 192), lambda b: (0, 0, 0)),
            pl.BlockSpec((1, 192), lambda b: (0, 0)),
            pl.BlockSpec((3, 192, 384), lambda b: (0, 0, 0)),
            pl.BlockSpec((1, 384), lambda b: (0, 0)),
        ],
        out_specs=pl.BlockSpec((1, _LA, _D_MODEL), lambda b: (b, 0, 0)),
        compiler_params=pltpu.CompilerParams(
            dimension_semantics=("parallel",),
            vmem_limit_bytes=_VMEM,
        ),
        name="audio_frontend",
    )(audio_t, w1t, b1, w2t, b2)


def _gather_kernel(ids_ref, embed_ref, o_ref, sem):
    b = pl.program_id(0)

    def body(i, carry):
        tok = ids_ref[b, i]
        pltpu.make_async_copy(embed_ref.at[pl.ds(tok, 1), :], o_ref.at[0, i],
                              sem).start()
        return carry

    jax.lax.fori_loop(0, _LT, body, 0)
    pltpu.make_async_copy(o_ref.at[0], o_ref.at[0], sem).wait()


def _embed_gather(ids, embed):
    return pl.pallas_call(
        _gather_kernel,
        out_shape=jax.ShapeDtypeStruct((_B, _LT, 1, _D_MODEL), jnp.float32),
        grid=(_B,),
        in_specs=[
            pl.BlockSpec(memory_space=pltpu.SMEM),
            pl.BlockSpec(memory_space=pl.ANY),
        ],
        out_specs=pl.BlockSpec((1, _LT, 1, _D_MODEL), lambda b: (b, 0, 0, 0)),
        scratch_shapes=[pltpu.SemaphoreType.DMA],
        compiler_params=pltpu.CompilerParams(
            dimension_semantics=("parallel",),
            vmem_limit_bytes=_VMEM,
        ),
        name="embed_gather",
    )(ids, embed)


def _bcast_state(v):
    return jax.lax.broadcast_in_dim(v, (v.shape[0], _D_STATE, _D_INNER), (0, 1))


def _mamba_kernel(x_ref, inwt_ref, ck_ref, cb_ref, xpdt_ref, xpb_ref, xpc_ref,
                  dtwt_ref, dtb_ref, alogt_ref, dp_ref, outwt_ref, o_ref,
                  h_ref, tail_ref, dA_ref, dBu_ref, H_ref):
    c = pl.program_id(1)

    @pl.when(c == 0)
    def _():
        h_ref[...] = jnp.zeros_like(h_ref)
        tail_ref[...] = jnp.zeros_like(tail_ref)

    x = x_ref[0]
    xz = jnp.dot(x, inwt_ref[...], preferred_element_type=jnp.float32)
    xi0 = xz[:, :_D_INNER]
    z = xz[:, _D_INNER:]

    prev = tail_ref[...]
    full = jnp.concatenate([prev[5:8], xi0], axis=0)
    tail_ref[...] = xi0[_T - 8:_T]
    xc = (full[0:_T] * ck_ref[0:1]
          + full[1:_T + 1] * ck_ref[1:2]
          + full[2:_T + 2] * ck_ref[2:3]
          + full[3:_T + 3] * ck_ref[3:4]) + cb_ref[...]
    xi = xc * jax.nn.sigmoid(xc)

    dtr = jnp.dot(xi, xpdt_ref[...], preferred_element_type=jnp.float32)
    bm = jnp.dot(xi, xpb_ref[...], preferred_element_type=jnp.float32)
    cm = jnp.dot(xi, xpc_ref[...], preferred_element_type=jnp.float32)
    dt = jax.nn.softplus(
        jnp.dot(dtr, dtwt_ref[...], preferred_element_type=jnp.float32)
        + dtb_ref[...])

    a_neg = -jnp.exp(alogt_ref[...])
    dt3 = dt.reshape(_T, 1, _D_INNER)
    dA_ref[...] = jnp.exp(dt3 * a_neg.reshape(1, _D_STATE, _D_INNER))
    u3 = (dt * xi).reshape(_T, 1, _D_INNER)
    dBu_ref[...] = u3 * _bcast_state(bm)

    def step(t, h):
        h = dA_ref[t] * h + dBu_ref[t]
        H_ref[t] = h
        return h

    h_ref[...] = jax.lax.fori_loop(0, _T, step, h_ref[...])

    ys = jnp.sum(H_ref[...] * _bcast_state(cm), axis=1)
    y = ys + xi * dp_ref[...]
    y = y * (z * jax.nn.sigmoid(z))
    o_ref[0] = jnp.dot(y, outwt_ref[...], preferred_element_type=jnp.float32)


def _mamba_layer(x, inwt, ck, cb, xpdt, xpb, xpc, dtwt, dtb, alogt, dp, outwt):
    return pl.pallas_call(
        _mamba_kernel,
        out_shape=jax.ShapeDtypeStruct((_B, _L, _D_MODEL), jnp.float32),
        grid=(_B, _NC),
        in_specs=[
            pl.BlockSpec((1, _T, _D_MODEL), lambda b, c: (b, c, 0)),
            pl.BlockSpec((_D_MODEL, 2 * _D_INNER), lambda b, c: (0, 0)),
            pl.BlockSpec((_D_CONV, _D_INNER), lambda b, c: (0, 0)),
            pl.BlockSpec((1, _D_INNER), lambda b, c: (0, 0)),
            pl.BlockSpec((_D_INNER, _DT_RANK), lambda b, c: (0, 0)),
            pl.BlockSpec((_D_INNER, _D_STATE), lambda b, c: (0, 0)),
            pl.BlockSpec((_D_INNER, _D_STATE), lambda b, c: (0, 0)),
            pl.BlockSpec((_DT_RANK, _D_INNER), lambda b, c: (0, 0)),
            pl.BlockSpec((1, _D_INNER), lambda b, c: (0, 0)),
            pl.BlockSpec((_D_STATE, _D_INNER), lambda b, c: (0, 0)),
            pl.BlockSpec((1, _D_INNER), lambda b, c: (0, 0)),
            pl.BlockSpec((_D_INNER, _D_MODEL), lambda b, c: (0, 0)),
        ],
        out_specs=pl.BlockSpec((1, _T, _D_MODEL), lambda b, c: (b, c, 0)),
        scratch_shapes=[
            pltpu.VMEM((_D_STATE, _D_INNER), jnp.float32),
            pltpu.VMEM((8, _D_INNER), jnp.float32),
            pltpu.VMEM((_T, _D_STATE, _D_INNER), jnp.float32),
            pltpu.VMEM((_T, _D_STATE, _D_INNER), jnp.float32),
            pltpu.VMEM((_T, _D_STATE, _D_INNER), jnp.float32),
        ],
        compiler_params=pltpu.CompilerParams(
            dimension_semantics=("parallel", "arbitrary"),
            vmem_limit_bytes=_VMEM,
        ),
        name="mamba_block",
    )(x, inwt, ck, cb, xpdt, xpb, xpc, dtwt, dtb, alogt, dp, outwt)


def _proj_kernel(x_ref, w_ref, b_ref, o_ref):
    xb = x_ref[...].reshape(_B * _L, _D_MODEL).astype(jnp.bfloat16)
    wb = w_ref[...].astype(jnp.bfloat16)
    acc = jax.lax.dot_general(
        xb, wb, (((1,), (1,)), ((), ())),
        preferred_element_type=jnp.float32)
    o_ref[...] = (acc + b_ref[...]).reshape(_B, _L, _NV)


def _final_proj(x3d, w, b2d):
    nv_tiles = (_VOCAB + _NV - 1) // _NV
    half = nv_tiles // 2
    return pl.pallas_call(
        _proj_kernel,
        out_shape=jax.ShapeDtypeStruct((_B, _L, _VOCAB), jnp.float32),
        grid=(2, half),
        in_specs=[
            pl.BlockSpec((_B, _L, _D_MODEL), lambda c, j: (0, 0, 0)),
            pl.BlockSpec((_NV, _D_MODEL), lambda c, j: (c * half + j, 0)),
            pl.BlockSpec((1, _NV), lambda c, j: (0, c * half + j)),
        ],
        out_specs=pl.BlockSpec((_B, _L, _NV), lambda c, j: (0, 0, c * half + j)),
        compiler_params=pltpu.CompilerParams(
            dimension_semantics=("parallel", "arbitrary"),
            vmem_limit_bytes=_VMEM,
        ),
        name="vocab_proj",
    )(x3d, w, b2d)


def _mamba_args(in_w, convw, convb, xproj_w, dt_w, dt_b, A_log, Dp, out_w):
    return (
        in_w.T,
        convw[:, 0, :].T,
        convb[None, :],
        xproj_w[:_DT_RANK].T,
        xproj_w[_DT_RANK:_DT_RANK + _D_STATE].T,
        xproj_w[_DT_RANK + _D_STATE:].T,
        dt_w.T,
        dt_b[None, :],
        A_log.T,
        Dp[None, :],
        out_w.T,
    )


def kernel(audio_features, text_tokens, conv1_w, conv1_b, conv2_w, conv2_b,
           embed, m1_in_w, m1_convw, m1_convb, m1_xproj_w, m1_dt_w, m1_dt_b,
           m1_A_log, m1_Dp, m1_out_w, m2_in_w, m2_convw, m2_convb, m2_xproj_w,
           m2_dt_w, m2_dt_b, m2_A_log, m2_Dp, m2_out_w, proj_w, proj_b):
    audio_t = audio_features.transpose(0, 2, 1)
    w1t = conv1_w.transpose(2, 1, 0)
    w2t = conv2_w.transpose(2, 1, 0)
    audio_emb = _audio_frontend(audio_t, w1t, conv1_b[None, :],
                                w2t, conv2_b[None, :])

    ids = text_tokens.astype(jnp.int32)
    text_emb = _embed_gather(ids, embed).reshape(_B, _LT, _D_MODEL)

    x = jnp.concatenate([audio_emb, text_emb], axis=1)

    x = _mamba_layer(x, *_mamba_args(m1_in_w, m1_convw, m1_convb, m1_xproj_w,
                                     m1_dt_w, m1_dt_b, m1_A_log, m1_Dp,
                                     m1_out_w))
    x = _mamba_layer(x, *_mamba_args(m2_in_w, m2_convw, m2_convb, m2_xproj_w,
                                     m2_dt_w, m2_dt_b, m2_A_log, m2_Dp,
                                     m2_out_w))

    return _final_proj(x, proj_w, proj_b[None, :])


# speedup vs baseline: 1.5238x; 1.5238x over previous
import jax
import jax.numpy as jnp
from jax.experimental import pallas as pl
from jax.experimental.pallas import tpu as pltpu

_D_MODEL = 384
_D_INNER = 768
_D_STATE = 16
_DT_RANK = 24
_D_CONV = 4
_VOCAB = 50257
_B = 2
_LA = 1024
_LT = 1024
_L = _LA + _LT

_T = 128
_NC = _L // _T
_NV = 1024
_VMEM = 56 * 1024 * 1024


def _audio_kernel(x_ref, w1_ref, b1_ref, w2_ref, b2_ref, o_ref):
    x = x_ref[0]
    zr1 = jnp.zeros((1, x.shape[1]), jnp.float32)
    xp = jnp.concatenate([zr1, x[:-1]], axis=0)
    xn = jnp.concatenate([x[1:], zr1], axis=0)
    y1 = (jnp.dot(xp, w1_ref[0], preferred_element_type=jnp.float32)
          + jnp.dot(x, w1_ref[1], preferred_element_type=jnp.float32)
          + jnp.dot(xn, w1_ref[2], preferred_element_type=jnp.float32))
    y1 = jnp.maximum(y1 + b1_ref[...], 0.0)
    zr2 = jnp.zeros((1, y1.shape[1]), jnp.float32)
    yp = jnp.concatenate([zr2, y1[:-1]], axis=0)
    yn = jnp.concatenate([y1[1:], zr2], axis=0)
    y2 = (jnp.dot(yp, w2_ref[0], preferred_element_type=jnp.float32)
          + jnp.dot(y1, w2_ref[1], preferred_element_type=jnp.float32)
          + jnp.dot(yn, w2_ref[2], preferred_element_type=jnp.float32))
    o_ref[0] = y2 + b2_ref[...]


def _audio_frontend(audio_t, w1t, b1, w2t, b2):
    return pl.pallas_call(
        _audio_kernel,
        out_shape=jax.ShapeDtypeStruct((_B, _LA, _D_MODEL), jnp.float32),
        grid=(_B,),
        in_specs=[
            pl.BlockSpec((1, _LA, 80), lambda b: (b, 0, 0)),
            pl.BlockSpec((3, 80, 192), lambda b: (0, 0, 0)),
            pl.BlockSpec((1, 192), lambda b: (0, 0)),
            pl.BlockSpec((3, 192, 384), lambda b: (0, 0, 0)),
            pl.BlockSpec((1, 384), lambda b: (0, 0)),
        ],
        out_specs=pl.BlockSpec((1, _LA, _D_MODEL), lambda b: (b, 0, 0)),
        compiler_params=pltpu.CompilerParams(
            dimension_semantics=("parallel",),
            vmem_limit_bytes=_VMEM,
        ),
        name="audio_frontend",
    )(audio_t, w1t, b1, w2t, b2)


def _gather_kernel(ids_ref, embed_ref, o_ref, sem):
    b = pl.program_id(0)

    def body(i, carry):
        tok = ids_ref[b, i]
        pltpu.make_async_copy(embed_ref.at[pl.ds(tok, 1), :], o_ref.at[0, i],
                              sem).start()
        return carry

    jax.lax.fori_loop(0, _LT, body, 0)
    pltpu.make_async_copy(o_ref.at[0], o_ref.at[0], sem).wait()


def _embed_gather(ids, embed):
    return pl.pallas_call(
        _gather_kernel,
        out_shape=jax.ShapeDtypeStruct((_B, _LT, 1, _D_MODEL), jnp.float32),
        grid=(_B,),
        in_specs=[
            pl.BlockSpec(memory_space=pltpu.SMEM),
            pl.BlockSpec(memory_space=pl.ANY),
        ],
        out_specs=pl.BlockSpec((1, _LT, 1, _D_MODEL), lambda b: (b, 0, 0, 0)),
        scratch_shapes=[pltpu.SemaphoreType.DMA],
        compiler_params=pltpu.CompilerParams(
            dimension_semantics=("parallel",),
            vmem_limit_bytes=_VMEM,
        ),
        name="embed_gather",
    )(ids, embed)


def _bcast_state(v):
    return jax.lax.broadcast_in_dim(v, (v.shape[0], _D_STATE, _D_INNER), (0, 1))


def _mamba_kernel(x_ref, inwt_ref, ck_ref, cb_ref, xpdt_ref, xpb_ref, xpc_ref,
                  dtwt_ref, dtb_ref, alogt_ref, dp_ref, outwt_ref, o_ref,
                  h_ref, tail_ref, dA_ref, dBu_ref, H_ref):
    c = pl.program_id(1)

    @pl.when(c == 0)
    def _():
        h_ref[...] = jnp.zeros_like(h_ref)
        tail_ref[...] = jnp.zeros_like(tail_ref)

    x = x_ref[0]
    xz = jnp.dot(x, inwt_ref[...], preferred_element_type=jnp.float32)
    xi0 = xz[:, :_D_INNER]
    z = xz[:, _D_INNER:]

    prev = tail_ref[...]
    full = jnp.concatenate([prev[5:8], xi0], axis=0)
    tail_ref[...] = xi0[_T - 8:_T]
    xc = (full[0:_T] * ck_ref[0:1]
          + full[1:_T + 1] * ck_ref[1:2]
          + full[2:_T + 2] * ck_ref[2:3]
          + full[3:_T + 3] * ck_ref[3:4]) + cb_ref[...]
    xi = xc * jax.nn.sigmoid(xc)

    dtr = jnp.dot(xi, xpdt_ref[...], preferred_element_type=jnp.float32)
    bm = jnp.dot(xi, xpb_ref[...], preferred_element_type=jnp.float32)
    cm = jnp.dot(xi, xpc_ref[...], preferred_element_type=jnp.float32)
    dt = jax.nn.softplus(
        jnp.dot(dtr, dtwt_ref[...], preferred_element_type=jnp.float32)
        + dtb_ref[...])

    a_neg = -jnp.exp(alogt_ref[...])
    dt3 = dt.reshape(_T, 1, _D_INNER)
    dA_ref[...] = jnp.exp(dt3 * a_neg.reshape(1, _D_STATE, _D_INNER))
    u3 = (dt * xi).reshape(_T, 1, _D_INNER)
    dBu_ref[...] = u3 * _bcast_state(bm)

    def step(t, h):
        h = dA_ref[t] * h + dBu_ref[t]
        H_ref[t] = h
        return h

    h_ref[...] = jax.lax.fori_loop(0, _T, step, h_ref[...])

    ys = jnp.sum(H_ref[...] * _bcast_state(cm), axis=1)
    y = ys + xi * dp_ref[...]
    y = y * (z * jax.nn.sigmoid(z))
    o_ref[0] = jnp.dot(y, outwt_ref[...], preferred_element_type=jnp.float32)


def _mamba_layer(x, inwt, ck, cb, xpdt, xpb, xpc, dtwt, dtb, alogt, dp, outwt):
    return pl.pallas_call(
        _mamba_kernel,
        out_shape=jax.ShapeDtypeStruct((_B, _L, _D_MODEL), jnp.float32),
        grid=(_B, _NC),
        in_specs=[
            pl.BlockSpec((1, _T, _D_MODEL), lambda b, c: (b, c, 0)),
            pl.BlockSpec((_D_MODEL, 2 * _D_INNER), lambda b, c: (0, 0)),
            pl.BlockSpec((_D_CONV, _D_INNER), lambda b, c: (0, 0)),
            pl.BlockSpec((1, _D_INNER), lambda b, c: (0, 0)),
            pl.BlockSpec((_D_INNER, _DT_RANK), lambda b, c: (0, 0)),
            pl.BlockSpec((_D_INNER, _D_STATE), lambda b, c: (0, 0)),
            pl.BlockSpec((_D_INNER, _D_STATE), lambda b, c: (0, 0)),
            pl.BlockSpec((_DT_RANK, _D_INNER), lambda b, c: (0, 0)),
            pl.BlockSpec((1, _D_INNER), lambda b, c: (0, 0)),
            pl.BlockSpec((_D_STATE, _D_INNER), lambda b, c: (0, 0)),
            pl.BlockSpec((1, _D_INNER), lambda b, c: (0, 0)),
            pl.BlockSpec((_D_INNER, _D_MODEL), lambda b, c: (0, 0)),
        ],
        out_specs=pl.BlockSpec((1, _T, _D_MODEL), lambda b, c: (b, c, 0)),
        scratch_shapes=[
            pltpu.VMEM((_D_STATE, _D_INNER), jnp.float32),
            pltpu.VMEM((8, _D_INNER), jnp.float32),
            pltpu.VMEM((_T, _D_STATE, _D_INNER), jnp.float32),
            pltpu.VMEM((_T, _D_STATE, _D_INNER), jnp.float32),
            pltpu.VMEM((_T, _D_STATE, _D_INNER), jnp.float32),
        ],
        compiler_params=pltpu.CompilerParams(
            dimension_semantics=("parallel", "arbitrary"),
            vmem_limit_bytes=_VMEM,
        ),
        name="mamba_block",
    )(x, inwt, ck, cb, xpdt, xpb, xpc, dtwt, dtb, alogt, dp, outwt)


def _proj_kernel(x_ref, w_ref, b_ref, o_ref):
    xb = x_ref[...].astype(jnp.bfloat16)
    wb = w_ref[...].astype(jnp.bfloat16)
    acc = jax.lax.dot_general(
        xb, wb, (((1,), (1,)), ((), ())),
        preferred_element_type=jnp.float32)
    o_ref[...] = acc + b_ref[...]


def _final_proj(x2d, w, b2d):
    nv_tiles = (_VOCAB + _NV - 1) // _NV
    return pl.pallas_call(
        _proj_kernel,
        out_shape=jax.ShapeDtypeStruct((_B * _L, _VOCAB), jnp.float32),
        grid=(nv_tiles,),
        in_specs=[
            pl.BlockSpec((_B * _L, _D_MODEL), lambda j: (0, 0)),
            pl.BlockSpec((_NV, _D_MODEL), lambda j: (j, 0)),
            pl.BlockSpec((1, _NV), lambda j: (0, j)),
        ],
        out_specs=pl.BlockSpec((_B * _L, _NV), lambda j: (0, j)),
        compiler_params=pltpu.CompilerParams(
            dimension_semantics=("parallel",),
            vmem_limit_bytes=_VMEM,
        ),
        name="vocab_proj",
    )(x2d, w, b2d)


def _mamba_args(in_w, convw, convb, xproj_w, dt_w, dt_b, A_log, Dp, out_w):
    return (
        in_w.T,
        convw[:, 0, :].T,
        convb[None, :],
        xproj_w[:_DT_RANK].T,
        xproj_w[_DT_RANK:_DT_RANK + _D_STATE].T,
        xproj_w[_DT_RANK + _D_STATE:].T,
        dt_w.T,
        dt_b[None, :],
        A_log.T,
        Dp[None, :],
        out_w.T,
    )


def kernel(audio_features, text_tokens, conv1_w, conv1_b, conv2_w, conv2_b,
           embed, m1_in_w, m1_convw, m1_convb, m1_xproj_w, m1_dt_w, m1_dt_b,
           m1_A_log, m1_Dp, m1_out_w, m2_in_w, m2_convw, m2_convb, m2_xproj_w,
           m2_dt_w, m2_dt_b, m2_A_log, m2_Dp, m2_out_w, proj_w, proj_b):
    audio_t = audio_features.transpose(0, 2, 1)
    w1t = conv1_w.transpose(2, 1, 0)
    w2t = conv2_w.transpose(2, 1, 0)
    audio_emb = _audio_frontend(audio_t, w1t, conv1_b[None, :],
                                w2t, conv2_b[None, :])

    ids = text_tokens.astype(jnp.int32)
    text_emb = _embed_gather(ids, embed).reshape(_B, _LT, _D_MODEL)

    x = jnp.concatenate([audio_emb, text_emb], axis=1)

    x = _mamba_layer(x, *_mamba_args(m1_in_w, m1_convw, m1_convb, m1_xproj_w,
                                     m1_dt_w, m1_dt_b, m1_A_log, m1_Dp,
                                     m1_out_w))
    x = _mamba_layer(x, *_mamba_args(m2_in_w, m2_convw, m2_convb, m2_xproj_w,
                                     m2_dt_w, m2_dt_b, m2_A_log, m2_Dp,
                                     m2_out_w))

    out = _final_proj(x.reshape(_B * _L, _D_MODEL), proj_w, proj_b[None, :])
    return out.reshape(_B, _L, _VOCAB)


# speedup vs baseline: 1.5686x; 1.0294x over previous
import jax
import jax.numpy as jnp
from jax.experimental import pallas as pl
from jax.experimental.pallas import tpu as pltpu

_D_MODEL = 384
_D_INNER = 768
_D_STATE = 16
_DT_RANK = 24
_D_CONV = 4
_VOCAB = 50257
_B = 2
_LA = 1024
_LT = 1024
_L = _LA + _LT

_T = 128
_NC = _L // _T
_NV = 1024
_VMEM = 56 * 1024 * 1024


def _audio_kernel(x_ref, w1_ref, b1_ref, w2_ref, b2_ref, o_ref):
    x = x_ref[0]
    zr1 = jnp.zeros((1, x.shape[1]), jnp.float32)
    xp = jnp.concatenate([zr1, x[:-1]], axis=0)
    xn = jnp.concatenate([x[1:], zr1], axis=0)
    y1 = (jnp.dot(xp, w1_ref[0], preferred_element_type=jnp.float32)
          + jnp.dot(x, w1_ref[1], preferred_element_type=jnp.float32)
          + jnp.dot(xn, w1_ref[2], preferred_element_type=jnp.float32))
    y1 = jnp.maximum(y1 + b1_ref[...], 0.0)
    zr2 = jnp.zeros((1, y1.shape[1]), jnp.float32)
    yp = jnp.concatenate([zr2, y1[:-1]], axis=0)
    yn = jnp.concatenate([y1[1:], zr2], axis=0)
    y2 = (jnp.dot(yp, w2_ref[0], preferred_element_type=jnp.float32)
          + jnp.dot(y1, w2_ref[1], preferred_element_type=jnp.float32)
          + jnp.dot(yn, w2_ref[2], preferred_element_type=jnp.float32))
    o_ref[0] = y2 + b2_ref[...]


def _audio_frontend(audio_t, w1t, b1, w2t, b2):
    return pl.pallas_call(
        _audio_kernel,
        out_shape=jax.ShapeDtypeStruct((_B, _LA, _D_MODEL), jnp.float32),
        grid=(_B,),
        in_specs=[
            pl.BlockSpec((1, _LA, 80), lambda b: (b, 0, 0)),
            pl.BlockSpec((3, 80, 192), lambda b: (0, 0, 0)),
            pl.BlockSpec((1, 192), lambda b: (0, 0)),
            pl.BlockSpec((3, 192, 384), lambda b: (0, 0, 0)),
            pl.BlockSpec((1, 384), lambda b: (0, 0)),
        ],
        out_specs=pl.BlockSpec((1, _LA, _D_MODEL), lambda b: (b, 0, 0)),
        compiler_params=pltpu.CompilerParams(
            dimension_semantics=("parallel",),
            vmem_limit_bytes=_VMEM,
        ),
        name="audio_frontend",
    )(audio_t, w1t, b1, w2t, b2)


def _gather_kernel(ids_ref, embed_ref, o_ref, sem):
    b = pl.program_id(0)

    def body(i, carry):
        tok = ids_ref[b, i]
        pltpu.make_async_copy(embed_ref.at[pl.ds(tok, 1), :], o_ref.at[0, i],
                              sem).start()
        return carry

    jax.lax.fori_loop(0, _LT, body, 0)
    pltpu.make_async_copy(o_ref.at[0], o_ref.at[0], sem).wait()


def _embed_gather(ids, embed):
    return pl.pallas_call(
        _gather_kernel,
        out_shape=jax.ShapeDtypeStruct((_B, _LT, 1, _D_MODEL), jnp.float32),
        grid=(_B,),
        in_specs=[
            pl.BlockSpec(memory_space=pltpu.SMEM),
            pl.BlockSpec(memory_space=pl.ANY),
        ],
        out_specs=pl.BlockSpec((1, _LT, 1, _D_MODEL), lambda b: (b, 0, 0, 0)),
        scratch_shapes=[pltpu.SemaphoreType.DMA],
        compiler_params=pltpu.CompilerParams(
            dimension_semantics=("parallel",),
            vmem_limit_bytes=_VMEM,
        ),
        name="embed_gather",
    )(ids, embed)


def _bcast_state(v):
    return jax.lax.broadcast_in_dim(v, (v.shape[0], _D_STATE, _D_INNER), (0, 1))


def _mamba_kernel(x_ref, inwt_ref, ck_ref, cb_ref, xpdt_ref, xpb_ref, xpc_ref,
                  dtwt_ref, dtb_ref, alogt_ref, dp_ref, outwt_ref, o_ref,
                  h_ref, tail_ref, dA_ref, dBu_ref, H_ref):
    c = pl.program_id(0)

    @pl.when(c == 0)
    def _():
        h_ref[...] = jnp.zeros_like(h_ref)
        tail_ref[...] = jnp.zeros_like(tail_ref)

    x = x_ref[...].reshape(_B * _T, _D_MODEL)
    xz = jnp.dot(x, inwt_ref[...], preferred_element_type=jnp.float32)
    xi0 = xz[:, :_D_INNER]
    z = xz[:, _D_INNER:]

    prev = tail_ref[...]
    xc_parts = []
    for b in range(_B):
        xi0_b = xi0[b * _T:(b + 1) * _T]
        full = jnp.concatenate([prev[8 * b + 5:8 * b + 8], xi0_b], axis=0)
        xc_parts.append(full[0:_T] * ck_ref[0:1]
                        + full[1:_T + 1] * ck_ref[1:2]
                        + full[2:_T + 2] * ck_ref[2:3]
                        + full[3:_T + 3] * ck_ref[3:4])
        tail_ref[8 * b:8 * (b + 1)] = xi0_b[_T - 8:_T]
    xc = jnp.concatenate(xc_parts, axis=0) + cb_ref[...]
    xi = xc * jax.nn.sigmoid(xc)

    dtr = jnp.dot(xi, xpdt_ref[...], preferred_element_type=jnp.float32)
    bm = jnp.dot(xi, xpb_ref[...], preferred_element_type=jnp.float32)
    cm = jnp.dot(xi, xpc_ref[...], preferred_element_type=jnp.float32)
    dt = jax.nn.softplus(
        jnp.dot(dtr, dtwt_ref[...], preferred_element_type=jnp.float32)
        + dtb_ref[...])

    a_neg = -jnp.exp(alogt_ref[...])
    dt3 = dt.reshape(_B * _T, 1, _D_INNER)
    dA_ref[...] = jnp.exp(dt3 * a_neg.reshape(1, _D_STATE, _D_INNER)
                          ).reshape(_B, _T, _D_STATE, _D_INNER)
    u3 = (dt * xi).reshape(_B * _T, 1, _D_INNER)
    dBu_ref[...] = (u3 * _bcast_state(bm)).reshape(_B, _T, _D_STATE, _D_INNER)

    def step(i, hs):
        h0, h1 = hs
        for u in range(2):
            t = 2 * i + u
            h0 = dA_ref[0, t] * h0 + dBu_ref[0, t]
            H_ref[0, t] = h0
            h1 = dA_ref[1, t] * h1 + dBu_ref[1, t]
            H_ref[1, t] = h1
        return (h0, h1)

    h0, h1 = jax.lax.fori_loop(
        0, _T // 2, step,
        (h_ref[0:_D_STATE], h_ref[_D_STATE:2 * _D_STATE]))
    h_ref[0:_D_STATE] = h0
    h_ref[_D_STATE:2 * _D_STATE] = h1

    Hv = H_ref[...].reshape(_B * _T, _D_STATE, _D_INNER)
    ys = jnp.sum(Hv * _bcast_state(cm), axis=1)
    y = ys + xi * dp_ref[...]
    y = y * (z * jax.nn.sigmoid(z))
    o_ref[...] = jnp.dot(y, outwt_ref[...],
                         preferred_element_type=jnp.float32
                         ).reshape(_B, _T, _D_MODEL)


def _mamba_layer(x, inwt, ck, cb, xpdt, xpb, xpc, dtwt, dtb, alogt, dp, outwt):
    return pl.pallas_call(
        _mamba_kernel,
        out_shape=jax.ShapeDtypeStruct((_B, _L, _D_MODEL), jnp.float32),
        grid=(_NC,),
        in_specs=[
            pl.BlockSpec((_B, _T, _D_MODEL), lambda c: (0, c, 0)),
            pl.BlockSpec((_D_MODEL, 2 * _D_INNER), lambda c: (0, 0)),
            pl.BlockSpec((_D_CONV, _D_INNER), lambda c: (0, 0)),
            pl.BlockSpec((1, _D_INNER), lambda c: (0, 0)),
            pl.BlockSpec((_D_INNER, _DT_RANK), lambda c: (0, 0)),
            pl.BlockSpec((_D_INNER, _D_STATE), lambda c: (0, 0)),
            pl.BlockSpec((_D_INNER, _D_STATE), lambda c: (0, 0)),
            pl.BlockSpec((_DT_RANK, _D_INNER), lambda c: (0, 0)),
            pl.BlockSpec((1, _D_INNER), lambda c: (0, 0)),
            pl.BlockSpec((_D_STATE, _D_INNER), lambda c: (0, 0)),
            pl.BlockSpec((1, _D_INNER), lambda c: (0, 0)),
            pl.BlockSpec((_D_INNER, _D_MODEL), lambda c: (0, 0)),
        ],
        out_specs=pl.BlockSpec((_B, _T, _D_MODEL), lambda c: (0, c, 0)),
        scratch_shapes=[
            pltpu.VMEM((_B * _D_STATE, _D_INNER), jnp.float32),
            pltpu.VMEM((_B * 8, _D_INNER), jnp.float32),
            pltpu.VMEM((_B, _T, _D_STATE, _D_INNER), jnp.float32),
            pltpu.VMEM((_B, _T, _D_STATE, _D_INNER), jnp.float32),
            pltpu.VMEM((_B, _T, _D_STATE, _D_INNER), jnp.float32),
        ],
        compiler_params=pltpu.CompilerParams(
            dimension_semantics=("arbitrary",),
            vmem_limit_bytes=_VMEM,
        ),
        name="mamba_block",
    )(x, inwt, ck, cb, xpdt, xpb, xpc, dtwt, dtb, alogt, dp, outwt)


def _proj_kernel(x_ref, w_ref, b_ref, o_ref):
    xb = x_ref[...].astype(jnp.bfloat16)
    wb = w_ref[...].astype(jnp.bfloat16)
    acc = jax.lax.dot_general(
        xb, wb, (((1,), (1,)), ((), ())),
        preferred_element_type=jnp.float32)
    o_ref[...] = acc + b_ref[...]


def _final_proj(x2d, w, b2d):
    nv_tiles = (_VOCAB + _NV - 1) // _NV
    return pl.pallas_call(
        _proj_kernel,
        out_shape=jax.ShapeDtypeStruct((_B * _L, _VOCAB), jnp.float32),
        grid=(nv_tiles,),
        in_specs=[
            pl.BlockSpec((_B * _L, _D_MODEL), lambda j: (0, 0)),
            pl.BlockSpec((_NV, _D_MODEL), lambda j: (j, 0)),
            pl.BlockSpec((1, _NV), lambda j: (0, j)),
        ],
        out_specs=pl.BlockSpec((_B * _L, _NV), lambda j: (0, j)),
        compiler_params=pltpu.CompilerParams(
            dimension_semantics=("parallel",),
            vmem_limit_bytes=_VMEM,
        ),
        name="vocab_proj",
    )(x2d, w, b2d)


def _mamba_args(in_w, convw, convb, xproj_w, dt_w, dt_b, A_log, Dp, out_w):
    return (
        in_w.T,
        convw[:, 0, :].T,
        convb[None, :],
        xproj_w[:_DT_RANK].T,
        xproj_w[_DT_RANK:_DT_RANK + _D_STATE].T,
        xproj_w[_DT_RANK + _D_STATE:].T,
        dt_w.T,
        dt_b[None, :],
        A_log.T,
        Dp[None, :],
        out_w.T,
    )


def kernel(audio_features, text_tokens, conv1_w, conv1_b, conv2_w, conv2_b,
           embed, m1_in_w, m1_convw, m1_convb, m1_xproj_w, m1_dt_w, m1_dt_b,
           m1_A_log, m1_Dp, m1_out_w, m2_in_w, m2_convw, m2_convb, m2_xproj_w,
           m2_dt_w, m2_dt_b, m2_A_log, m2_Dp, m2_out_w, proj_w, proj_b):
    audio_t = audio_features.transpose(0, 2, 1)
    w1t = conv1_w.transpose(2, 1, 0)
    w2t = conv2_w.transpose(2, 1, 0)
    audio_emb = _audio_frontend(audio_t, w1t, conv1_b[None, :],
                                w2t, conv2_b[None, :])

    ids = text_tokens.astype(jnp.int32)
    text_emb = _embed_gather(ids, embed).reshape(_B, _LT, _D_MODEL)

    x = jnp.concatenate([audio_emb, text_emb], axis=1)

    x = _mamba_layer(x, *_mamba_args(m1_in_w, m1_convw, m1_convb, m1_xproj_w,
                                     m1_dt_w, m1_dt_b, m1_A_log, m1_Dp,
                                     m1_out_w))
    x = _mamba_layer(x, *_mamba_args(m2_in_w, m2_convw, m2_convb, m2_xproj_w,
                                     m2_dt_w, m2_dt_b, m2_A_log, m2_Dp,
                                     m2_out_w))

    out = _final_proj(x.reshape(_B * _L, _D_MODEL), proj_w, proj_b[None, :])
    return out.reshape(_B, _L, _VOCAB)


# speedup vs baseline: 1.5753x; 1.0043x over previous
import jax
import jax.numpy as jnp
from jax.experimental import pallas as pl
from jax.experimental.pallas import tpu as pltpu

_D_MODEL = 384
_D_INNER = 768
_D_STATE = 16
_DT_RANK = 24
_D_CONV = 4
_VOCAB = 50257
_B = 2
_LA = 1024
_LT = 1024
_L = _LA + _LT

_T = 128
_NC = _L // _T
_NV = 1024
_VMEM = 56 * 1024 * 1024


def _audio_kernel(x_ref, w1_ref, b1_ref, w2_ref, b2_ref, o_ref):
    x = x_ref[0]
    zr1 = jnp.zeros((1, x.shape[1]), jnp.float32)
    xp = jnp.concatenate([zr1, x[:-1]], axis=0)
    xn = jnp.concatenate([x[1:], zr1], axis=0)
    y1 = (jnp.dot(xp, w1_ref[0], preferred_element_type=jnp.float32)
          + jnp.dot(x, w1_ref[1], preferred_element_type=jnp.float32)
          + jnp.dot(xn, w1_ref[2], preferred_element_type=jnp.float32))
    y1 = jnp.maximum(y1 + b1_ref[...], 0.0)
    zr2 = jnp.zeros((1, y1.shape[1]), jnp.float32)
    yp = jnp.concatenate([zr2, y1[:-1]], axis=0)
    yn = jnp.concatenate([y1[1:], zr2], axis=0)
    y2 = (jnp.dot(yp, w2_ref[0], preferred_element_type=jnp.float32)
          + jnp.dot(y1, w2_ref[1], preferred_element_type=jnp.float32)
          + jnp.dot(yn, w2_ref[2], preferred_element_type=jnp.float32))
    o_ref[0] = y2 + b2_ref[...]


def _audio_frontend(audio_t, w1t, b1, w2t, b2):
    return pl.pallas_call(
        _audio_kernel,
        out_shape=jax.ShapeDtypeStruct((_B, _LA, _D_MODEL), jnp.float32),
        grid=(_B,),
        in_specs=[
            pl.BlockSpec((1, _LA, 80), lambda b: (b, 0, 0)),
            pl.BlockSpec((3, 80, 192), lambda b: (0, 0, 0)),
            pl.BlockSpec((1, 192), lambda b: (0, 0)),
            pl.BlockSpec((3, 192, 384), lambda b: (0, 0, 0)),
            pl.BlockSpec((1, 384), lambda b: (0, 0)),
        ],
        out_specs=pl.BlockSpec((1, _LA, _D_MODEL), lambda b: (b, 0, 0)),
        compiler_params=pltpu.CompilerParams(
            dimension_semantics=("parallel",),
            vmem_limit_bytes=_VMEM,
        ),
        name="audio_frontend",
    )(audio_t, w1t, b1, w2t, b2)


def _gather_kernel(ids_ref, embed_ref, o_ref, sem):
    b = pl.program_id(0)

    def body(i, carry):
        tok = ids_ref[b, i]
        pltpu.make_async_copy(embed_ref.at[pl.ds(tok, 1), :], o_ref.at[0, i],
                              sem).start()
        return carry

    jax.lax.fori_loop(0, _LT, body, 0)
    pltpu.make_async_copy(o_ref.at[0], o_ref.at[0], sem).wait()


def _embed_gather(ids, embed):
    return pl.pallas_call(
        _gather_kernel,
        out_shape=jax.ShapeDtypeStruct((_B, _LT, 1, _D_MODEL), jnp.float32),
        grid=(_B,),
        in_specs=[
            pl.BlockSpec(memory_space=pltpu.SMEM),
            pl.BlockSpec(memory_space=pl.ANY),
        ],
        out_specs=pl.BlockSpec((1, _LT, 1, _D_MODEL), lambda b: (b, 0, 0, 0)),
        scratch_shapes=[pltpu.SemaphoreType.DMA],
        compiler_params=pltpu.CompilerParams(
            dimension_semantics=("parallel",),
            vmem_limit_bytes=_VMEM,
        ),
        name="embed_gather",
    )(ids, embed)


def _bcast_state(v):
    small = jax.lax.broadcast_in_dim(v, (v.shape[0], _D_STATE, 128), (0, 1))
    return pltpu.repeat(small, _D_INNER // 128, axis=2)


def _softplus(x):
    return jnp.maximum(x, 0.0) + jnp.log1p(jnp.exp(-jnp.abs(x)))


def _mamba_kernel(x_ref, inwt_ref, ck_ref, cb_ref, xpdt_ref, xpb_ref, xpc_ref,
                  dtwt_ref, dtb_ref, alogt_ref, dp_ref, outwt_ref, o_ref,
                  h_ref, tail_ref, dAB_ref, H_ref):
    c = pl.program_id(0)

    @pl.when(c == 0)
    def _():
        h_ref[...] = jnp.zeros_like(h_ref)
        tail_ref[...] = jnp.zeros_like(tail_ref)

    x = x_ref[...].reshape(_B * _T, _D_MODEL)
    xz = jnp.dot(x, inwt_ref[...], preferred_element_type=jnp.float32)
    xi0 = xz[:, :_D_INNER]
    z = xz[:, _D_INNER:]

    prev = tail_ref[...]
    xc_parts = []
    for b in range(_B):
        xi0_b = xi0[b * _T:(b + 1) * _T]
        full = jnp.concatenate([prev[8 * b + 5:8 * b + 8], xi0_b], axis=0)
        xc_parts.append(full[0:_T] * ck_ref[0:1]
                        + full[1:_T + 1] * ck_ref[1:2]
                        + full[2:_T + 2] * ck_ref[2:3]
                        + full[3:_T + 3] * ck_ref[3:4])
        tail_ref[8 * b:8 * (b + 1)] = xi0_b[_T - 8:_T]
    xc = jnp.concatenate(xc_parts, axis=0) + cb_ref[...]
    xi = xc * jax.nn.sigmoid(xc)

    dtr = jnp.dot(xi, xpdt_ref[...], preferred_element_type=jnp.float32)
    bm = jnp.dot(xi, xpb_ref[...], preferred_element_type=jnp.float32)
    cm = jnp.dot(xi, xpc_ref[...], preferred_element_type=jnp.float32)
    dt = _softplus(
        jnp.dot(dtr, dtwt_ref[...], preferred_element_type=jnp.float32)
        + dtb_ref[...])

    a_neg = -jnp.exp(alogt_ref[...])
    dt3 = dt.reshape(_B * _T, 1, _D_INNER)
    dAB_ref[:, :, 0:_D_STATE, :] = jnp.exp(
        dt3 * a_neg.reshape(1, _D_STATE, _D_INNER)
    ).reshape(_B, _T, _D_STATE, _D_INNER)
    u3 = (dt * xi).reshape(_B * _T, 1, _D_INNER)
    dAB_ref[:, :, _D_STATE:2 * _D_STATE, :] = (
        u3 * _bcast_state(bm)).reshape(_B, _T, _D_STATE, _D_INNER)

    def step(i, hs):
        h0, h1 = hs
        for u in range(4):
            t = 4 * i + u
            ab0 = dAB_ref[0, t]
            h0 = ab0[0:_D_STATE] * h0 + ab0[_D_STATE:]
            H_ref[0, t] = h0
            ab1 = dAB_ref[1, t]
            h1 = ab1[0:_D_STATE] * h1 + ab1[_D_STATE:]
            H_ref[1, t] = h1
        return (h0, h1)

    h0, h1 = jax.lax.fori_loop(
        0, _T // 4, step,
        (h_ref[0:_D_STATE], h_ref[_D_STATE:2 * _D_STATE]))
    h_ref[0:_D_STATE] = h0
    h_ref[_D_STATE:2 * _D_STATE] = h1

    Hv = H_ref[...].reshape(_B * _T, _D_STATE, _D_INNER)
    ys = jnp.sum(Hv * _bcast_state(cm), axis=1)
    y = ys + xi * dp_ref[...]
    y = y * (z * jax.nn.sigmoid(z))
    o_ref[...] = jnp.dot(y, outwt_ref[...],
                         preferred_element_type=jnp.float32
                         ).reshape(_B, _T, _D_MODEL)


def _mamba_layer(x, inwt, ck, cb, xpdt, xpb, xpc, dtwt, dtb, alogt, dp, outwt):
    return pl.pallas_call(
        _mamba_kernel,
        out_shape=jax.ShapeDtypeStruct((_B, _L, _D_MODEL), jnp.float32),
        grid=(_NC,),
        in_specs=[
            pl.BlockSpec((_B, _T, _D_MODEL), lambda c: (0, c, 0)),
            pl.BlockSpec((_D_MODEL, 2 * _D_INNER), lambda c: (0, 0)),
            pl.BlockSpec((_D_CONV, _D_INNER), lambda c: (0, 0)),
            pl.BlockSpec((1, _D_INNER), lambda c: (0, 0)),
            pl.BlockSpec((_D_INNER, _DT_RANK), lambda c: (0, 0)),
            pl.BlockSpec((_D_INNER, _D_STATE), lambda c: (0, 0)),
            pl.BlockSpec((_D_INNER, _D_STATE), lambda c: (0, 0)),
            pl.BlockSpec((_DT_RANK, _D_INNER), lambda c: (0, 0)),
            pl.BlockSpec((1, _D_INNER), lambda c: (0, 0)),
            pl.BlockSpec((_D_STATE, _D_INNER), lambda c: (0, 0)),
            pl.BlockSpec((1, _D_INNER), lambda c: (0, 0)),
            pl.BlockSpec((_D_INNER, _D_MODEL), lambda c: (0, 0)),
        ],
        out_specs=pl.BlockSpec((_B, _T, _D_MODEL), lambda c: (0, c, 0)),
        scratch_shapes=[
            pltpu.VMEM((_B * _D_STATE, _D_INNER), jnp.float32),
            pltpu.VMEM((_B * 8, _D_INNER), jnp.float32),
            pltpu.VMEM((_B, _T, 2 * _D_STATE, _D_INNER), jnp.float32),
            pltpu.VMEM((_B, _T, _D_STATE, _D_INNER), jnp.float32),
        ],
        compiler_params=pltpu.CompilerParams(
            dimension_semantics=("arbitrary",),
            vmem_limit_bytes=_VMEM,
        ),
        name="mamba_block",
    )(x, inwt, ck, cb, xpdt, xpb, xpc, dtwt, dtb, alogt, dp, outwt)


def _proj_kernel(x_ref, w_ref, b_ref, o_ref):
    xb = x_ref[...].astype(jnp.bfloat16)
    wb = w_ref[...].astype(jnp.bfloat16)
    acc = jax.lax.dot_general(
        xb, wb, (((1,), (1,)), ((), ())),
        preferred_element_type=jnp.float32)
    o_ref[...] = acc + b_ref[...]


def _final_proj(x2d, w, b2d):
    nv_tiles = (_VOCAB + _NV - 1) // _NV
    return pl.pallas_call(
        _proj_kernel,
        out_shape=jax.ShapeDtypeStruct((_B * _L, _VOCAB), jnp.float32),
        grid=(nv_tiles,),
        in_specs=[
            pl.BlockSpec((_B * _L, _D_MODEL), lambda j: (0, 0)),
            pl.BlockSpec((_NV, _D_MODEL), lambda j: (j, 0)),
            pl.BlockSpec((1, _NV), lambda j: (0, j)),
        ],
        out_specs=pl.BlockSpec((_B * _L, _NV), lambda j: (0, j)),
        compiler_params=pltpu.CompilerParams(
            dimension_semantics=("parallel",),
            vmem_limit_bytes=_VMEM,
        ),
        name="vocab_proj",
    )(x2d, w, b2d)


def _mamba_args(in_w, convw, convb, xproj_w, dt_w, dt_b, A_log, Dp, out_w):
    return (
        in_w.T,
        convw[:, 0, :].T,
        convb[None, :],
        xproj_w[:_DT_RANK].T,
        xproj_w[_DT_RANK:_DT_RANK + _D_STATE].T,
        xproj_w[_DT_RANK + _D_STATE:].T,
        dt_w.T,
        dt_b[None, :],
        A_log.T,
        Dp[None, :],
        out_w.T,
    )


def kernel(audio_features, text_tokens, conv1_w, conv1_b, conv2_w, conv2_b,
           embed, m1_in_w, m1_convw, m1_convb, m1_xproj_w, m1_dt_w, m1_dt_b,
           m1_A_log, m1_Dp, m1_out_w, m2_in_w, m2_convw, m2_convb, m2_xproj_w,
           m2_dt_w, m2_dt_b, m2_A_log, m2_Dp, m2_out_w, proj_w, proj_b):
    audio_t = audio_features.transpose(0, 2, 1)
    w1t = conv1_w.transpose(2, 1, 0)
    w2t = conv2_w.transpose(2, 1, 0)
    audio_emb = _audio_frontend(audio_t, w1t, conv1_b[None, :],
                                w2t, conv2_b[None, :])

    ids = text_tokens.astype(jnp.int32)
    text_emb = _embed_gather(ids, embed).reshape(_B, _LT, _D_MODEL)

    x = jnp.concatenate([audio_emb, text_emb], axis=1)

    x = _mamba_layer(x, *_mamba_args(m1_in_w, m1_convw, m1_convb, m1_xproj_w,
                                     m1_dt_w, m1_dt_b, m1_A_log, m1_Dp,
                                     m1_out_w))
    x = _mamba_layer(x, *_mamba_args(m2_in_w, m2_convw, m2_convb, m2_xproj_w,
                                     m2_dt_w, m2_dt_b, m2_A_log, m2_Dp,
                                     m2_out_w))

    out = _final_proj(x.reshape(_B * _L, _D_MODEL), proj_w, proj_b[None, :])
    return out.reshape(_B, _L, _VOCAB)


# speedup vs baseline: 1.5796x; 1.0027x over previous
import jax
import jax.numpy as jnp
from jax.experimental import pallas as pl
from jax.experimental.pallas import tpu as pltpu

_D_MODEL = 384
_D_INNER = 768
_D_STATE = 16
_DT_RANK = 24
_D_CONV = 4
_VOCAB = 50257
_B = 2
_LA = 1024
_LT = 1024
_L = _LA + _LT

_T = 128
_NC = _L // _T
_NV = 1024
_VMEM = 56 * 1024 * 1024


def _audio_kernel(x_ref, w1_ref, b1_ref, w2_ref, b2_ref, o_ref):
    x = x_ref[0]
    zr1 = jnp.zeros((1, x.shape[1]), jnp.float32)
    xp = jnp.concatenate([zr1, x[:-1]], axis=0)
    xn = jnp.concatenate([x[1:], zr1], axis=0)
    y1 = (jnp.dot(xp, w1_ref[0], preferred_element_type=jnp.float32)
          + jnp.dot(x, w1_ref[1], preferred_element_type=jnp.float32)
          + jnp.dot(xn, w1_ref[2], preferred_element_type=jnp.float32))
    y1 = jnp.maximum(y1 + b1_ref[...], 0.0)
    zr2 = jnp.zeros((1, y1.shape[1]), jnp.float32)
    yp = jnp.concatenate([zr2, y1[:-1]], axis=0)
    yn = jnp.concatenate([y1[1:], zr2], axis=0)
    y2 = (jnp.dot(yp, w2_ref[0], preferred_element_type=jnp.float32)
          + jnp.dot(y1, w2_ref[1], preferred_element_type=jnp.float32)
          + jnp.dot(yn, w2_ref[2], preferred_element_type=jnp.float32))
    o_ref[0] = y2 + b2_ref[...]


def _audio_frontend(audio_t, w1t, b1, w2t, b2):
    return pl.pallas_call(
        _audio_kernel,
        out_shape=jax.ShapeDtypeStruct((_B, _LA, _D_MODEL), jnp.float32),
        grid=(_B,),
        in_specs=[
            pl.BlockSpec((1, _LA, 80), lambda b: (b, 0, 0)),
            pl.BlockSpec((3, 80, 192), lambda b: (0, 0, 0)),
            pl.BlockSpec((1, 192), lambda b: (0, 0)),
            pl.BlockSpec((3, 192, 384), lambda b: (0, 0, 0)),
            pl.BlockSpec((1, 384), lambda b: (0, 0)),
        ],
        out_specs=pl.BlockSpec((1, _LA, _D_MODEL), lambda b: (b, 0, 0)),
        compiler_params=pltpu.CompilerParams(
            dimension_semantics=("parallel",),
            vmem_limit_bytes=_VMEM,
        ),
        name="audio_frontend",
    )(audio_t, w1t, b1, w2t, b2)


def _gather_kernel(ids_ref, embed_ref, o_ref, sem):
    b = pl.program_id(0)

    def body(i, carry):
        tok = ids_ref[b, i]
        pltpu.make_async_copy(embed_ref.at[pl.ds(tok, 1), :], o_ref.at[0, i],
                              sem).start()
        return carry

    jax.lax.fori_loop(0, _LT, body, 0)
    pltpu.make_async_copy(o_ref.at[0], o_ref.at[0], sem).wait()


def _embed_gather(ids, embed):
    return pl.pallas_call(
        _gather_kernel,
        out_shape=jax.ShapeDtypeStruct((_B, _LT, 1, _D_MODEL), jnp.float32),
        grid=(_B,),
        in_specs=[
            pl.BlockSpec(memory_space=pltpu.SMEM),
            pl.BlockSpec(memory_space=pl.ANY),
        ],
        out_specs=pl.BlockSpec((1, _LT, 1, _D_MODEL), lambda b: (b, 0, 0, 0)),
        scratch_shapes=[pltpu.SemaphoreType.DMA],
        compiler_params=pltpu.CompilerParams(
            dimension_semantics=("parallel",),
            vmem_limit_bytes=_VMEM,
        ),
        name="embed_gather",
    )(ids, embed)


def _bcast_state(v):
    small = jax.lax.broadcast_in_dim(v, (v.shape[0], _D_STATE, 128), (0, 1))
    return pltpu.repeat(small, _D_INNER // 128, axis=2)


def _softplus(x):
    return jnp.maximum(x, 0.0) + jnp.log1p(jnp.exp(-jnp.abs(x)))


def _mamba_body(x, inwt_ref, ck_ref, cb_ref, xpdt_ref, xpb_ref, xpc_ref,
                dtwt_ref, dtb_ref, alogt_ref, dp_ref, outwt_ref,
                h_ref, tail_ref, dAB_ref, H_ref):
    xz = jnp.dot(x, inwt_ref[...], preferred_element_type=jnp.float32)
    xi0 = xz[:, :_D_INNER]
    z = xz[:, _D_INNER:]

    prev = tail_ref[...]
    xc_parts = []
    for b in range(_B):
        xi0_b = xi0[b * _T:(b + 1) * _T]
        full = jnp.concatenate([prev[8 * b + 5:8 * b + 8], xi0_b], axis=0)
        xc_parts.append(full[0:_T] * ck_ref[0:1]
                        + full[1:_T + 1] * ck_ref[1:2]
                        + full[2:_T + 2] * ck_ref[2:3]
                        + full[3:_T + 3] * ck_ref[3:4])
        tail_ref[8 * b:8 * (b + 1)] = xi0_b[_T - 8:_T]
    xc = jnp.concatenate(xc_parts, axis=0) + cb_ref[...]
    xi = xc * jax.nn.sigmoid(xc)

    dtr = jnp.dot(xi, xpdt_ref[...], preferred_element_type=jnp.float32)
    bm = jnp.dot(xi, xpb_ref[...], preferred_element_type=jnp.float32)
    cm = jnp.dot(xi, xpc_ref[...], preferred_element_type=jnp.float32)
    dt = _softplus(
        jnp.dot(dtr, dtwt_ref[...], preferred_element_type=jnp.float32)
        + dtb_ref[...])

    a_neg = -jnp.exp(alogt_ref[...])
    dt3 = dt.reshape(_B * _T, 1, _D_INNER)
    dAB_ref[:, :, 0:_D_STATE, :] = jnp.exp(
        dt3 * a_neg.reshape(1, _D_STATE, _D_INNER)
    ).reshape(_B, _T, _D_STATE, _D_INNER)
    u3 = (dt * xi).reshape(_B * _T, 1, _D_INNER)
    dAB_ref[:, :, _D_STATE:2 * _D_STATE, :] = (
        u3 * _bcast_state(bm)).reshape(_B, _T, _D_STATE, _D_INNER)

    def step(i, hs):
        h0, h1 = hs
        for u in range(4):
            t = 4 * i + u
            ab0 = dAB_ref[0, t]
            h0 = ab0[0:_D_STATE] * h0 + ab0[_D_STATE:]
            H_ref[0, t] = h0
            ab1 = dAB_ref[1, t]
            h1 = ab1[0:_D_STATE] * h1 + ab1[_D_STATE:]
            H_ref[1, t] = h1
        return (h0, h1)

    h0, h1 = jax.lax.fori_loop(
        0, _T // 4, step,
        (h_ref[0:_D_STATE], h_ref[_D_STATE:2 * _D_STATE]))
    h_ref[0:_D_STATE] = h0
    h_ref[_D_STATE:2 * _D_STATE] = h1

    Hv = H_ref[...].reshape(_B * _T, _D_STATE, _D_INNER)
    ys = jnp.sum(Hv * _bcast_state(cm), axis=1)
    y = ys + xi * dp_ref[...]
    y = y * (z * jax.nn.sigmoid(z))
    return jnp.dot(y, outwt_ref[...], preferred_element_type=jnp.float32)


def _mamba2_kernel(x_ref, *refs):
    w1 = refs[0:11]
    w2 = refs[11:22]
    o_ref = refs[22]
    h1_ref, t1_ref, h2_ref, t2_ref, dAB_ref, H_ref = refs[23:29]
    c = pl.program_id(0)

    @pl.when(c == 0)
    def _():
        h1_ref[...] = jnp.zeros_like(h1_ref)
        t1_ref[...] = jnp.zeros_like(t1_ref)
        h2_ref[...] = jnp.zeros_like(h2_ref)
        t2_ref[...] = jnp.zeros_like(t2_ref)

    x = x_ref[...].reshape(_B * _T, _D_MODEL)
    x1 = _mamba_body(x, *w1, h1_ref, t1_ref, dAB_ref, H_ref)
    x2 = _mamba_body(x1, *w2, h2_ref, t2_ref, dAB_ref, H_ref)
    o_ref[...] = x2.reshape(_B, _T, _D_MODEL)


def _mamba_layers(x, w1, w2):
    wspec = [
        pl.BlockSpec((_D_MODEL, 2 * _D_INNER), lambda c: (0, 0)),
        pl.BlockSpec((_D_CONV, _D_INNER), lambda c: (0, 0)),
        pl.BlockSpec((1, _D_INNER), lambda c: (0, 0)),
        pl.BlockSpec((_D_INNER, _DT_RANK), lambda c: (0, 0)),
        pl.BlockSpec((_D_INNER, _D_STATE), lambda c: (0, 0)),
        pl.BlockSpec((_D_INNER, _D_STATE), lambda c: (0, 0)),
        pl.BlockSpec((_DT_RANK, _D_INNER), lambda c: (0, 0)),
        pl.BlockSpec((1, _D_INNER), lambda c: (0, 0)),
        pl.BlockSpec((_D_STATE, _D_INNER), lambda c: (0, 0)),
        pl.BlockSpec((1, _D_INNER), lambda c: (0, 0)),
        pl.BlockSpec((_D_INNER, _D_MODEL), lambda c: (0, 0)),
    ]
    return pl.pallas_call(
        _mamba2_kernel,
        out_shape=jax.ShapeDtypeStruct((_B, _L, _D_MODEL), jnp.float32),
        grid=(_NC,),
        in_specs=[pl.BlockSpec((_B, _T, _D_MODEL), lambda c: (0, c, 0))]
        + wspec + wspec,
        out_specs=pl.BlockSpec((_B, _T, _D_MODEL), lambda c: (0, c, 0)),
        scratch_shapes=[
            pltpu.VMEM((_B * _D_STATE, _D_INNER), jnp.float32),
            pltpu.VMEM((_B * 8, _D_INNER), jnp.float32),
            pltpu.VMEM((_B * _D_STATE, _D_INNER), jnp.float32),
            pltpu.VMEM((_B * 8, _D_INNER), jnp.float32),
            pltpu.VMEM((_B, _T, 2 * _D_STATE, _D_INNER), jnp.float32),
            pltpu.VMEM((_B, _T, _D_STATE, _D_INNER), jnp.float32),
        ],
        compiler_params=pltpu.CompilerParams(
            dimension_semantics=("arbitrary",),
            vmem_limit_bytes=_VMEM,
        ),
        name="mamba_block",
    )(x, *w1, *w2)


def _proj_kernel(x_ref, w_ref, b_ref, o_ref):
    xb = x_ref[...].astype(jnp.bfloat16)
    wb = w_ref[...].astype(jnp.bfloat16)
    acc = jax.lax.dot_general(
        xb, wb, (((1,), (1,)), ((), ())),
        preferred_element_type=jnp.float32)
    o_ref[...] = acc + b_ref[...]


def _final_proj(x2d, w, b2d):
    nv_tiles = (_VOCAB + _NV - 1) // _NV
    return pl.pallas_call(
        _proj_kernel,
        out_shape=jax.ShapeDtypeStruct((_B * _L, _VOCAB), jnp.float32),
        grid=(nv_tiles,),
        in_specs=[
            pl.BlockSpec((_B * _L, _D_MODEL), lambda j: (0, 0)),
            pl.BlockSpec((_NV, _D_MODEL), lambda j: (j, 0)),
            pl.BlockSpec((1, _NV), lambda j: (0, j)),
        ],
        out_specs=pl.BlockSpec((_B * _L, _NV), lambda j: (0, j)),
        compiler_params=pltpu.CompilerParams(
            dimension_semantics=("parallel",),
            vmem_limit_bytes=_VMEM,
        ),
        name="vocab_proj",
    )(x2d, w, b2d)


def _mamba_args(in_w, convw, convb, xproj_w, dt_w, dt_b, A_log, Dp, out_w):
    return (
        in_w.T,
        convw[:, 0, :].T,
        convb[None, :],
        xproj_w[:_DT_RANK].T,
        xproj_w[_DT_RANK:_DT_RANK + _D_STATE].T,
        xproj_w[_DT_RANK + _D_STATE:].T,
        dt_w.T,
        dt_b[None, :],
        A_log.T,
        Dp[None, :],
        out_w.T,
    )


def kernel(audio_features, text_tokens, conv1_w, conv1_b, conv2_w, conv2_b,
           embed, m1_in_w, m1_convw, m1_convb, m1_xproj_w, m1_dt_w, m1_dt_b,
           m1_A_log, m1_Dp, m1_out_w, m2_in_w, m2_convw, m2_convb, m2_xproj_w,
           m2_dt_w, m2_dt_b, m2_A_log, m2_Dp, m2_out_w, proj_w, proj_b):
    audio_t = audio_features.transpose(0, 2, 1)
    w1t = conv1_w.transpose(2, 1, 0)
    w2t = conv2_w.transpose(2, 1, 0)
    audio_emb = _audio_frontend(audio_t, w1t, conv1_b[None, :],
                                w2t, conv2_b[None, :])

    ids = text_tokens.astype(jnp.int32)
    text_emb = _embed_gather(ids, embed).reshape(_B, _LT, _D_MODEL)

    x = jnp.concatenate([audio_emb, text_emb], axis=1)

    x = _mamba_layers(
        x,
        _mamba_args(m1_in_w, m1_convw, m1_convb, m1_xproj_w, m1_dt_w,
                    m1_dt_b, m1_A_log, m1_Dp, m1_out_w),
        _mamba_args(m2_in_w, m2_convw, m2_convb, m2_xproj_w, m2_dt_w,
                    m2_dt_b, m2_A_log, m2_Dp, m2_out_w))

    out = _final_proj(x.reshape(_B * _L, _D_MODEL), proj_w, proj_b[None, :])
    return out.reshape(_B, _L, _VOCAB)


# speedup vs baseline: 1.5829x; 1.0021x over previous
import jax
import jax.numpy as jnp
from jax.experimental import pallas as pl
from jax.experimental.pallas import tpu as pltpu

_D_MODEL = 384
_D_INNER = 768
_D_STATE = 16
_DT_RANK = 24
_D_CONV = 4
_VOCAB = 50257
_B = 2
_LA = 1024
_LT = 1024
_L = _LA + _LT

_T = 128
_NC = _L // _T
_NV = 1024
_VMEM = 56 * 1024 * 1024


def _audio_kernel(x_ref, w1_ref, b1_ref, w2_ref, b2_ref, o_ref):
    x = x_ref[0]
    zr1 = jnp.zeros((1, x.shape[1]), jnp.float32)
    xp = jnp.concatenate([zr1, x[:-1]], axis=0)
    xn = jnp.concatenate([x[1:], zr1], axis=0)
    y1 = (jnp.dot(xp, w1_ref[0], preferred_element_type=jnp.float32)
          + jnp.dot(x, w1_ref[1], preferred_element_type=jnp.float32)
          + jnp.dot(xn, w1_ref[2], preferred_element_type=jnp.float32))
    y1 = jnp.maximum(y1 + b1_ref[...], 0.0)
    zr2 = jnp.zeros((1, y1.shape[1]), jnp.float32)
    yp = jnp.concatenate([zr2, y1[:-1]], axis=0)
    yn = jnp.concatenate([y1[1:], zr2], axis=0)
    y2 = (jnp.dot(yp, w2_ref[0], preferred_element_type=jnp.float32)
          + jnp.dot(y1, w2_ref[1], preferred_element_type=jnp.float32)
          + jnp.dot(yn, w2_ref[2], preferred_element_type=jnp.float32))
    o_ref[0] = y2 + b2_ref[...]


def _audio_frontend(audio_t, w1t, b1, w2t, b2):
    return pl.pallas_call(
        _audio_kernel,
        out_shape=jax.ShapeDtypeStruct((_B, _LA, _D_MODEL), jnp.float32),
        grid=(_B,),
        in_specs=[
            pl.BlockSpec((1, _LA, 80), lambda b: (b, 0, 0)),
            pl.BlockSpec((3, 80, 192), lambda b: (0, 0, 0)),
            pl.BlockSpec((1, 192), lambda b: (0, 0)),
            pl.BlockSpec((3, 192, 384), lambda b: (0, 0, 0)),
            pl.BlockSpec((1, 384), lambda b: (0, 0)),
        ],
        out_specs=pl.BlockSpec((1, _LA, _D_MODEL), lambda b: (b, 0, 0)),
        compiler_params=pltpu.CompilerParams(
            dimension_semantics=("parallel",),
            vmem_limit_bytes=_VMEM,
        ),
        name="audio_frontend",
    )(audio_t, w1t, b1, w2t, b2)


def _gather_kernel(ids_ref, embed_ref, o_ref, sem):
    b = pl.program_id(0)

    def body(i, carry):
        for u in range(8):
            tok = ids_ref[b, 8 * i + u]
            pltpu.make_async_copy(embed_ref.at[pl.ds(tok, 1), :],
                                  o_ref.at[0, 8 * i + u], sem).start()
        return carry

    jax.lax.fori_loop(0, _LT // 8, body, 0)
    pltpu.make_async_copy(o_ref.at[0], o_ref.at[0], sem).wait()


def _embed_gather(ids, embed):
    return pl.pallas_call(
        _gather_kernel,
        out_shape=jax.ShapeDtypeStruct((_B, _LT, 1, _D_MODEL), jnp.float32),
        grid=(_B,),
        in_specs=[
            pl.BlockSpec(memory_space=pltpu.SMEM),
            pl.BlockSpec(memory_space=pl.ANY),
        ],
        out_specs=pl.BlockSpec((1, _LT, 1, _D_MODEL), lambda b: (b, 0, 0, 0)),
        scratch_shapes=[pltpu.SemaphoreType.DMA],
        compiler_params=pltpu.CompilerParams(
            dimension_semantics=("parallel",),
            vmem_limit_bytes=_VMEM,
        ),
        name="embed_gather",
    )(ids, embed)


def _bcast_state(v):
    small = jax.lax.broadcast_in_dim(v, (v.shape[0], _D_STATE, 128), (0, 1))
    return pltpu.repeat(small, _D_INNER // 128, axis=2)


def _softplus(x):
    return jnp.maximum(x, 0.0) + jnp.log1p(jnp.exp(-jnp.abs(x)))


def _mamba_body(x, inwt_ref, ck_ref, cb_ref, xpdt_ref, xpb_ref, xpc_ref,
                dtwt_ref, dtb_ref, alogt_ref, dp_ref, outwt_ref,
                h_ref, tail_ref, dAB_ref, H_ref):
    xz = jnp.dot(x, inwt_ref[...], preferred_element_type=jnp.float32)
    xi0 = xz[:, :_D_INNER]
    z = xz[:, _D_INNER:]

    prev = tail_ref[...]
    xc_parts = []
    for b in range(_B):
        xi0_b = xi0[b * _T:(b + 1) * _T]
        full = jnp.concatenate([prev[8 * b + 5:8 * b + 8], xi0_b], axis=0)
        xc_parts.append(full[0:_T] * ck_ref[0:1]
                        + full[1:_T + 1] * ck_ref[1:2]
                        + full[2:_T + 2] * ck_ref[2:3]
                        + full[3:_T + 3] * ck_ref[3:4])
        tail_ref[8 * b:8 * (b + 1)] = xi0_b[_T - 8:_T]
    xc = jnp.concatenate(xc_parts, axis=0) + cb_ref[...]
    xi = xc * jax.nn.sigmoid(xc)

    dtr = jnp.dot(xi, xpdt_ref[...], preferred_element_type=jnp.float32)
    bm = jnp.dot(xi, xpb_ref[...], preferred_element_type=jnp.float32)
    cm = jnp.dot(xi, xpc_ref[...], preferred_element_type=jnp.float32)
    dt = _softplus(
        jnp.dot(dtr, dtwt_ref[...], preferred_element_type=jnp.float32)
        + dtb_ref[...])

    a_neg = -jnp.exp(alogt_ref[...])
    dt3 = dt.reshape(_B * _T, 1, _D_INNER)
    dAB_ref[:, :, 0:_D_STATE, :] = jnp.exp(
        dt3 * a_neg.reshape(1, _D_STATE, _D_INNER)
    ).reshape(_B, _T, _D_STATE, _D_INNER)
    u3 = (dt * xi).reshape(_B * _T, 1, _D_INNER)
    dAB_ref[:, :, _D_STATE:2 * _D_STATE, :] = (
        u3 * _bcast_state(bm)).reshape(_B, _T, _D_STATE, _D_INNER)

    def step(i, hs):
        h0, h1 = hs
        for u in range(4):
            t = 4 * i + u
            ab0 = dAB_ref[0, t]
            h0 = ab0[0:_D_STATE] * h0 + ab0[_D_STATE:]
            H_ref[0, t] = h0
            ab1 = dAB_ref[1, t]
            h1 = ab1[0:_D_STATE] * h1 + ab1[_D_STATE:]
            H_ref[1, t] = h1
        return (h0, h1)

    h0, h1 = jax.lax.fori_loop(
        0, _T // 4, step,
        (h_ref[0:_D_STATE], h_ref[_D_STATE:2 * _D_STATE]))
    h_ref[0:_D_STATE] = h0
    h_ref[_D_STATE:2 * _D_STATE] = h1

    Hv = H_ref[...].reshape(_B * _T, _D_STATE, _D_INNER)
    ys = jnp.sum(Hv * _bcast_state(cm), axis=1)
    y = ys + xi * dp_ref[...]
    y = y * (z * jax.nn.sigmoid(z))
    return jnp.dot(y, outwt_ref[...], preferred_element_type=jnp.float32)


def _mamba2_kernel(x_ref, *refs):
    w1 = refs[0:11]
    w2 = refs[11:22]
    o_ref = refs[22]
    h1_ref, t1_ref, h2_ref, t2_ref, dAB_ref, H_ref = refs[23:29]
    c = pl.program_id(0)

    @pl.when(c == 0)
    def _():
        h1_ref[...] = jnp.zeros_like(h1_ref)
        t1_ref[...] = jnp.zeros_like(t1_ref)
        h2_ref[...] = jnp.zeros_like(h2_ref)
        t2_ref[...] = jnp.zeros_like(t2_ref)

    x = x_ref[...].reshape(_B * _T, _D_MODEL)
    x1 = _mamba_body(x, *w1, h1_ref, t1_ref, dAB_ref, H_ref)
    x2 = _mamba_body(x1, *w2, h2_ref, t2_ref, dAB_ref, H_ref)
    o_ref[...] = x2.reshape(_B, _T, _D_MODEL)


def _mamba_layers(x, w1, w2):
    wspec = [
        pl.BlockSpec((_D_MODEL, 2 * _D_INNER), lambda c: (0, 0)),
        pl.BlockSpec((_D_CONV, _D_INNER), lambda c: (0, 0)),
        pl.BlockSpec((1, _D_INNER), lambda c: (0, 0)),
        pl.BlockSpec((_D_INNER, _DT_RANK), lambda c: (0, 0)),
        pl.BlockSpec((_D_INNER, _D_STATE), lambda c: (0, 0)),
        pl.BlockSpec((_D_INNER, _D_STATE), lambda c: (0, 0)),
        pl.BlockSpec((_DT_RANK, _D_INNER), lambda c: (0, 0)),
        pl.BlockSpec((1, _D_INNER), lambda c: (0, 0)),
        pl.BlockSpec((_D_STATE, _D_INNER), lambda c: (0, 0)),
        pl.BlockSpec((1, _D_INNER), lambda c: (0, 0)),
        pl.BlockSpec((_D_INNER, _D_MODEL), lambda c: (0, 0)),
    ]
    return pl.pallas_call(
        _mamba2_kernel,
        out_shape=jax.ShapeDtypeStruct((_B, _L, _D_MODEL), jnp.float32),
        grid=(_NC,),
        in_specs=[pl.BlockSpec((_B, _T, _D_MODEL), lambda c: (0, c, 0))]
        + wspec + wspec,
        out_specs=pl.BlockSpec((_B, _T, _D_MODEL), lambda c: (0, c, 0)),
        scratch_shapes=[
            pltpu.VMEM((_B * _D_STATE, _D_INNER), jnp.float32),
            pltpu.VMEM((_B * 8, _D_INNER), jnp.float32),
            pltpu.VMEM((_B * _D_STATE, _D_INNER), jnp.float32),
            pltpu.VMEM((_B * 8, _D_INNER), jnp.float32),
            pltpu.VMEM((_B, _T, 2 * _D_STATE, _D_INNER), jnp.float32),
            pltpu.VMEM((_B, _T, _D_STATE, _D_INNER), jnp.float32),
        ],
        compiler_params=pltpu.CompilerParams(
            dimension_semantics=("arbitrary",),
            vmem_limit_bytes=_VMEM,
        ),
        name="mamba_block",
    )(x, *w1, *w2)


def _proj_kernel(x_ref, w_ref, b_ref, o_ref):
    xb = x_ref[...].astype(jnp.bfloat16)
    wb = w_ref[...].astype(jnp.bfloat16)
    acc = jax.lax.dot_general(
        xb, wb, (((1,), (1,)), ((), ())),
        preferred_element_type=jnp.float32)
    o_ref[...] = acc + b_ref[...]


def _final_proj(x2d, w, b2d):
    nv_tiles = (_VOCAB + _NV - 1) // _NV
    return pl.pallas_call(
        _proj_kernel,
        out_shape=jax.ShapeDtypeStruct((_B * _L, _VOCAB), jnp.float32),
        grid=(nv_tiles,),
        in_specs=[
            pl.BlockSpec((_B * _L, _D_MODEL), lambda j: (0, 0)),
            pl.BlockSpec((_NV, _D_MODEL), lambda j: (j, 0)),
            pl.BlockSpec((1, _NV), lambda j: (0, j)),
        ],
        out_specs=pl.BlockSpec((_B * _L, _NV), lambda j: (0, j)),
        compiler_params=pltpu.CompilerParams(
            dimension_semantics=("parallel",),
            vmem_limit_bytes=_VMEM,
        ),
        name="vocab_proj",
    )(x2d, w, b2d)


def _mamba_args(in_w, convw, convb, xproj_w, dt_w, dt_b, A_log, Dp, out_w):
    return (
        in_w.T,
        convw[:, 0, :].T,
        convb[None, :],
        xproj_w[:_DT_RANK].T,
        xproj_w[_DT_RANK:_DT_RANK + _D_STATE].T,
        xproj_w[_DT_RANK + _D_STATE:].T,
        dt_w.T,
        dt_b[None, :],
        A_log.T,
        Dp[None, :],
        out_w.T,
    )


def kernel(audio_features, text_tokens, conv1_w, conv1_b, conv2_w, conv2_b,
           embed, m1_in_w, m1_convw, m1_convb, m1_xproj_w, m1_dt_w, m1_dt_b,
           m1_A_log, m1_Dp, m1_out_w, m2_in_w, m2_convw, m2_convb, m2_xproj_w,
           m2_dt_w, m2_dt_b, m2_A_log, m2_Dp, m2_out_w, proj_w, proj_b):
    audio_t = audio_features.transpose(0, 2, 1)
    w1t = conv1_w.transpose(2, 1, 0)
    w2t = conv2_w.transpose(2, 1, 0)
    audio_emb = _audio_frontend(audio_t, w1t, conv1_b[None, :],
                                w2t, conv2_b[None, :])

    ids = text_tokens.astype(jnp.int32)
    text_emb = _embed_gather(ids, embed).reshape(_B, _LT, _D_MODEL)

    x = jnp.concatenate([audio_emb, text_emb], axis=1)

    x = _mamba_layers(
        x,
        _mamba_args(m1_in_w, m1_convw, m1_convb, m1_xproj_w, m1_dt_w,
                    m1_dt_b, m1_A_log, m1_Dp, m1_out_w),
        _mamba_args(m2_in_w, m2_convw, m2_convb, m2_xproj_w, m2_dt_w,
                    m2_dt_b, m2_A_log, m2_Dp, m2_out_w))

    out = _final_proj(x.reshape(_B * _L, _D_MODEL), proj_w, proj_b[None, :])
    return out.reshape(_B, _L, _VOCAB)
